# baseline (speedup 1.0000x reference)
import functools
import math

import jax
import jax.numpy as jnp
from jax import lax
from jax.experimental import pallas as pl
from jax.experimental.pallas import tpu as pltpu
from jax.experimental.pallas import tpu_sc as plsc

_PAD = 0
_SMOOTHING = 0.1
_CONF = 1.0 - _SMOOTHING

_BR = 16


def _tc_body(x_ref, w_ref, out_ref):
    i = pl.program_id(0)
    xb = x_ref[...]
    w = w_ref[...]
    s = jnp.sum(xb, axis=1, keepdims=True)
    s = s - xb[:, 0:1]
    part = jnp.sum(s * w).reshape(1, 1)

    @pl.when(i == 0)
    def _init():
        out_ref[...] = jnp.zeros((1, 1), jnp.float32)

    out_ref[...] += part


def _tc_rowsum(x2, w):
    n, v = x2.shape
    grid = n // _BR
    out = pl.pallas_call(
        _tc_body,
        grid=(grid,),
        in_specs=[
            pl.BlockSpec((_BR, v), lambda i: (i, 0)),
            pl.BlockSpec((_BR, 1), lambda i: (i, 0)),
        ],
        out_specs=pl.BlockSpec((1, 1), lambda i: (0, 0)),
        out_shape=jax.ShapeDtypeStruct((1, 1), jnp.float32),
    )(x2, w)
    return out[0, 0]


@functools.lru_cache(maxsize=None)
def _sc_gather_build(n, v):
    info = plsc.get_sparse_core_info()
    nc, ns, L = info.num_cores, info.num_subcores, info.num_lanes
    nw = nc * ns
    bpw = n // nw
    mesh = plsc.VectorSubcoreMesh(core_axis_name="c", subcore_axis_name="s")

    @functools.partial(
        pl.kernel,
        mesh=mesh,
        out_type=[
            jax.ShapeDtypeStruct((nw, L), jnp.float32),
            jax.ShapeDtypeStruct((nw, L), jnp.float32),
        ],
        scratch_types=[
            pltpu.VMEM((bpw,), jnp.int32),
            pltpu.VMEM((bpw,), jnp.int32),
            pltpu.VMEM((bpw,), jnp.float32),
            pltpu.VMEM((L,), jnp.float32),
            pltpu.VMEM((L,), jnp.float32),
            pltpu.SemaphoreType.DMA,
        ],
    )
    def sc_gather(xflat_hbm, tgt_hbm, gsum_hbm, cnt_hbm,
                  tgt_v, idx_v, val_v, g_v, c_v, sem):
        wid = lax.axis_index("s") * nc + lax.axis_index("c")
        base = wid * bpw
        pltpu.sync_copy(tgt_hbm.at[pl.ds(base, bpw)], tgt_v)
        for j in range(bpw // L):
            t16 = tgt_v[pl.ds(j * L, L)]
            rows = base + j * L + lax.iota(jnp.int32, L)
            idx_v[pl.ds(j * L, L)] = rows * v + t16
        pltpu.async_copy(xflat_hbm.at[idx_v], val_v, sem).wait()
        g = jnp.zeros((L,), jnp.float32)
        c = jnp.zeros((L,), jnp.float32)
        for j in range(bpw // L):
            t16 = tgt_v[pl.ds(j * L, L)]
            v16 = val_v[pl.ds(j * L, L)]
            m = t16 != _PAD
            g = g + jnp.where(m, v16, 0.0)
            c = c + jnp.where(m, 1.0, 0.0)
        g_v[...] = g
        c_v[...] = c
        pltpu.sync_copy(g_v, gsum_hbm.at[wid])
        pltpu.sync_copy(c_v, cnt_hbm.at[wid])

    return sc_gather


def kernel(x, target):
    v = x.shape[-1]
    x2 = x.reshape(-1, v)
    n = x2.shape[0]
    t = target.reshape(-1)

    eps = _SMOOTHING / (v - 2)
    c_row = (v - 2) * eps * math.log(eps) + _CONF * math.log(_CONF)

    w = (t != _PAD).astype(jnp.float32).reshape(n, 1)

    gpart, cpart = _sc_gather_build(n, v)(x2.reshape(-1), t)
    tcsum = _tc_rowsum(x2, w)

    gsum = jnp.sum(gpart)
    nval = jnp.sum(cpart)
    return nval * c_row - eps * tcsum - (_CONF - eps) * gsum


# speedup vs baseline: 2.3279x; 2.3279x over previous
import functools
import math

import jax
import jax.numpy as jnp
from jax import lax
from jax.experimental import pallas as pl
from jax.experimental.pallas import tpu as pltpu
from jax.experimental.pallas import tpu_sc as plsc

_PAD = 0
_SMOOTHING = 0.1
_CONF = 1.0 - _SMOOTHING

_BR = 16


def _tc_body(x_ref, w_ref, out_ref):
    i = pl.program_id(0)
    xb = x_ref[...]
    w = w_ref[...]
    s = jnp.sum(xb, axis=1, keepdims=True)
    s = s - xb[:, 0:1]
    part = jnp.sum(s * w).reshape(1, 1)

    @pl.when(i == 0)
    def _init():
        out_ref[...] = jnp.zeros((1, 1), jnp.float32)

    out_ref[...] += part


def _tc_rowsum(x2, w):
    n, v = x2.shape
    grid = n // _BR
    out = pl.pallas_call(
        _tc_body,
        grid=(grid,),
        in_specs=[
            pl.BlockSpec((_BR, v), lambda i: (i, 0)),
            pl.BlockSpec((_BR, 1), lambda i: (i, 0)),
        ],
        out_specs=pl.BlockSpec((1, 1), lambda i: (0, 0)),
        out_shape=jax.ShapeDtypeStruct((1, 1), jnp.float32),
    )(x2, w)
    return out[0, 0]


@functools.lru_cache(maxsize=None)
def _sc_gather_build(n, v):
    info = plsc.get_sparse_core_info()
    nc, ns, L = info.num_cores, info.num_subcores, info.num_lanes
    nw = nc * ns
    bpw = n // nw
    mesh = plsc.VectorSubcoreMesh(core_axis_name="c", subcore_axis_name="s")

    @functools.partial(
        pl.kernel,
        mesh=mesh,
        out_type=[
            jax.ShapeDtypeStruct((nw, L), jnp.float32),
            jax.ShapeDtypeStruct((nw, L), jnp.float32),
        ],
        scratch_types=[
            pltpu.VMEM((bpw,), jnp.int32),
            pltpu.VMEM((8, 128), jnp.float32),
            pltpu.VMEM((L,), jnp.float32),
            pltpu.VMEM((L,), jnp.float32),
        ],
    )
    def sc_gather(x_hbm, tgt_hbm, gsum_hbm, cnt_hbm,
                  tgt_v, buf_v, g_v, c_v):
        wid = lax.axis_index("s") * nc + lax.axis_index("c")
        base = wid * bpw
        pltpu.sync_copy(tgt_hbm.at[pl.ds(base, bpw)], tgt_v)
        lane = lax.iota(jnp.int32, L)
        g = jnp.zeros((L,), jnp.float32)
        c = jnp.zeros((L,), jnp.float32)
        for j in range(bpw // L):
            t16 = tgt_v[pl.ds(j * L, L)]
            c = c + jnp.where(t16 != _PAD, 1.0, 0.0)
        for i in range(bpw):
            t16 = tgt_v[pl.ds((i // L) * L, L)]
            ti = t16[i % L]
            ctile = pl.multiple_of((ti >> 7) << 7, 128)
            rtile = pl.multiple_of(base + (i // 8) * 8, 8)
            pltpu.sync_copy(
                x_hbm.at[pl.ds(rtile, 8), pl.ds(ctile, 128)], buf_v)
            cseg = ((ti >> 4) << 4) - ctile
            seg = buf_v[i % 8, pl.ds(cseg, L)]
            hit = jnp.where(lane == (ti & (L - 1)), seg, 0.0)
            validf = jnp.minimum(ti, 1).astype(jnp.float32)
            g = g + hit * validf
        g_v[...] = g
        c_v[...] = c
        pltpu.sync_copy(g_v, gsum_hbm.at[wid])
        pltpu.sync_copy(c_v, cnt_hbm.at[wid])

    return sc_gather


def kernel(x, target):
    v = x.shape[-1]
    x2 = x.reshape(-1, v)
    n = x2.shape[0]
    t = target.reshape(-1)

    eps = _SMOOTHING / (v - 2)
    c_row = (v - 2) * eps * math.log(eps) + _CONF * math.log(_CONF)

    w = (t != _PAD).astype(jnp.float32).reshape(n, 1)

    gpart, cpart = _sc_gather_build(n, v)(x2, t)
    tcsum = _tc_rowsum(x2, w)

    gsum = jnp.sum(gpart)
    nval = jnp.sum(cpart)
    return nval * c_row - eps * tcsum - (_CONF - eps) * gsum


# speedup vs baseline: 2.3880x; 1.0258x over previous
import functools
import math

import jax
import jax.numpy as jnp
from jax import lax
from jax.experimental import pallas as pl
from jax.experimental.pallas import tpu as pltpu
from jax.experimental.pallas import tpu_sc as plsc

_PAD = 0
_SMOOTHING = 0.1
_CONF = 1.0 - _SMOOTHING

_BR = 32


def _tc_body(x_ref, w_ref, out_ref):
    i = pl.program_id(0)
    xb = x_ref[...]
    w = w_ref[...]
    s = jnp.sum(xb, axis=1, keepdims=True)
    s = s - xb[:, 0:1]
    part = jnp.sum(s * w).reshape(1, 1)

    @pl.when(i == 0)
    def _init():
        out_ref[...] = jnp.zeros((1, 1), jnp.float32)

    out_ref[...] += part


def _tc_rowsum(x2, w):
    n, v = x2.shape
    grid = n // _BR
    out = pl.pallas_call(
        _tc_body,
        grid=(grid,),
        in_specs=[
            pl.BlockSpec((_BR, v), lambda i: (i, 0)),
            pl.BlockSpec((_BR, 1), lambda i: (i, 0)),
        ],
        out_specs=pl.BlockSpec((1, 1), lambda i: (0, 0)),
        out_shape=jax.ShapeDtypeStruct((1, 1), jnp.float32),
    )(x2, w)
    return out[0, 0]


@functools.lru_cache(maxsize=None)
def _sc_gather_build(n, v):
    info = plsc.get_sparse_core_info()
    nc, ns, L = info.num_cores, info.num_subcores, info.num_lanes
    nw = nc * ns
    bpw = n // nw
    mesh = plsc.VectorSubcoreMesh(core_axis_name="c", subcore_axis_name="s")

    @functools.partial(
        pl.kernel,
        mesh=mesh,
        out_type=[
            jax.ShapeDtypeStruct((nw, L), jnp.float32),
            jax.ShapeDtypeStruct((nw, L), jnp.float32),
        ],
        scratch_types=[
            pltpu.VMEM((bpw,), jnp.int32),
            pltpu.VMEM((8, 128), jnp.float32),
            pltpu.VMEM((L,), jnp.float32),
            pltpu.VMEM((L,), jnp.float32),
        ],
    )
    def sc_gather(x_hbm, tgt_hbm, gsum_hbm, cnt_hbm,
                  tgt_v, buf_v, g_v, c_v):
        wid = lax.axis_index("s") * nc + lax.axis_index("c")
        base = wid * bpw
        pltpu.sync_copy(tgt_hbm.at[pl.ds(base, bpw)], tgt_v)
        lane = lax.iota(jnp.int32, L)
        g = jnp.zeros((L,), jnp.float32)
        c = jnp.zeros((L,), jnp.float32)
        for j in range(bpw // L):
            t16 = tgt_v[pl.ds(j * L, L)]
            c = c + jnp.where(t16 != _PAD, 1.0, 0.0)
        for i in range(bpw):
            t16 = tgt_v[pl.ds((i // L) * L, L)]
            ti = t16[i % L]
            ctile = pl.multiple_of((ti >> 7) << 7, 128)
            rtile = pl.multiple_of(base + (i // 8) * 8, 8)
            pltpu.sync_copy(
                x_hbm.at[pl.ds(rtile, 8), pl.ds(ctile, 128)], buf_v)
            cseg = ((ti >> 4) << 4) - ctile
            seg = buf_v[i % 8, pl.ds(cseg, L)]
            hit = jnp.where(lane == (ti & (L - 1)), seg, 0.0)
            validf = jnp.minimum(ti, 1).astype(jnp.float32)
            g = g + hit * validf
        g_v[...] = g
        c_v[...] = c
        pltpu.sync_copy(g_v, gsum_hbm.at[wid])
        pltpu.sync_copy(c_v, cnt_hbm.at[wid])

    return sc_gather


def kernel(x, target):
    v = x.shape[-1]
    x2 = x.reshape(-1, v)
    n = x2.shape[0]
    t = target.reshape(-1)

    eps = _SMOOTHING / (v - 2)
    c_row = (v - 2) * eps * math.log(eps) + _CONF * math.log(_CONF)

    w = (t != _PAD).astype(jnp.float32).reshape(n, 1)

    gpart, cpart = _sc_gather_build(n, v)(x2, t)
    tcsum = _tc_rowsum(x2, w)

    gsum = jnp.sum(gpart)
    nval = jnp.sum(cpart)
    return nval * c_row - eps * tcsum - (_CONF - eps) * gsum


# speedup vs baseline: 2.3892x; 1.0005x over previous
import functools
import math

import jax
import jax.numpy as jnp
from jax import lax
from jax.experimental import pallas as pl
from jax.experimental.pallas import tpu as pltpu
from jax.experimental.pallas import tpu_sc as plsc

_PAD = 0
_SMOOTHING = 0.1
_CONF = 1.0 - _SMOOTHING

_BR = 64


def _tc_body(x_ref, w_ref, out_ref):
    i = pl.program_id(0)
    xb = x_ref[...]
    w = w_ref[...]
    s = jnp.sum(xb, axis=1, keepdims=True)
    s = s - xb[:, 0:1]
    part = jnp.sum(s * w).reshape(1, 1)

    @pl.when(i == 0)
    def _init():
        out_ref[...] = jnp.zeros((1, 1), jnp.float32)

    out_ref[...] += part


def _tc_rowsum(x2, w):
    n, v = x2.shape
    grid = n // _BR
    out = pl.pallas_call(
        _tc_body,
        grid=(grid,),
        in_specs=[
            pl.BlockSpec((_BR, v), lambda i: (i, 0)),
            pl.BlockSpec((_BR, 1), lambda i: (i, 0)),
        ],
        out_specs=pl.BlockSpec((1, 1), lambda i: (0, 0)),
        out_shape=jax.ShapeDtypeStruct((1, 1), jnp.float32),
    )(x2, w)
    return out[0, 0]


@functools.lru_cache(maxsize=None)
def _sc_gather_build(n, v):
    info = plsc.get_sparse_core_info()
    nc, ns, L = info.num_cores, info.num_subcores, info.num_lanes
    nw = nc * ns
    bpw = n // nw
    mesh = plsc.VectorSubcoreMesh(core_axis_name="c", subcore_axis_name="s")

    @functools.partial(
        pl.kernel,
        mesh=mesh,
        out_type=[
            jax.ShapeDtypeStruct((nw, L), jnp.float32),
            jax.ShapeDtypeStruct((nw, L), jnp.float32),
        ],
        scratch_types=[
            pltpu.VMEM((bpw,), jnp.int32),
            pltpu.VMEM((8, 128), jnp.float32),
            pltpu.VMEM((L,), jnp.float32),
            pltpu.VMEM((L,), jnp.float32),
        ],
    )
    def sc_gather(x_hbm, tgt_hbm, gsum_hbm, cnt_hbm,
                  tgt_v, buf_v, g_v, c_v):
        wid = lax.axis_index("s") * nc + lax.axis_index("c")
        base = wid * bpw
        pltpu.sync_copy(tgt_hbm.at[pl.ds(base, bpw)], tgt_v)
        lane = lax.iota(jnp.int32, L)
        g = jnp.zeros((L,), jnp.float32)
        c = jnp.zeros((L,), jnp.float32)
        for j in range(bpw // L):
            t16 = tgt_v[pl.ds(j * L, L)]
            c = c + jnp.where(t16 != _PAD, 1.0, 0.0)
        for i in range(bpw):
            t16 = tgt_v[pl.ds((i // L) * L, L)]
            ti = t16[i % L]
            ctile = pl.multiple_of((ti >> 7) << 7, 128)
            rtile = pl.multiple_of(base + (i // 8) * 8, 8)
            pltpu.sync_copy(
                x_hbm.at[pl.ds(rtile, 8), pl.ds(ctile, 128)], buf_v)
            cseg = ((ti >> 4) << 4) - ctile
            seg = buf_v[i % 8, pl.ds(cseg, L)]
            hit = jnp.where(lane == (ti & (L - 1)), seg, 0.0)
            validf = jnp.minimum(ti, 1).astype(jnp.float32)
            g = g + hit * validf
        g_v[...] = g
        c_v[...] = c
        pltpu.sync_copy(g_v, gsum_hbm.at[wid])
        pltpu.sync_copy(c_v, cnt_hbm.at[wid])

    return sc_gather


def kernel(x, target):
    v = x.shape[-1]
    x2 = x.reshape(-1, v)
    n = x2.shape[0]
    t = target.reshape(-1)

    eps = _SMOOTHING / (v - 2)
    c_row = (v - 2) * eps * math.log(eps) + _CONF * math.log(_CONF)

    w = (t != _PAD).astype(jnp.float32).reshape(n, 1)

    gpart, cpart = _sc_gather_build(n, v)(x2, t)
    tcsum = _tc_rowsum(x2, w)

    gsum = jnp.sum(gpart)
    nval = jnp.sum(cpart)
    return nval * c_row - eps * tcsum - (_CONF - eps) * gsum


# speedup vs baseline: 2.3945x; 1.0022x over previous
import functools
import math

import jax
import jax.numpy as jnp
from jax import lax
from jax.experimental import pallas as pl
from jax.experimental.pallas import tpu as pltpu
from jax.experimental.pallas import tpu_sc as plsc

_PAD = 0
_SMOOTHING = 0.1
_CONF = 1.0 - _SMOOTHING

_BR = 64


def _tc_body(x_ref, w_ref, out_ref):
    i = pl.program_id(0)
    xb = x_ref[...]
    w = w_ref[...]
    s = jnp.sum(xb, axis=1, keepdims=True)
    s = s - xb[:, 0:1]
    part = jnp.sum(s * w).reshape(1, 1)

    @pl.when(i == 0)
    def _init():
        out_ref[...] = jnp.zeros((1, 1), jnp.float32)

    out_ref[...] += part


def _tc_rowsum(x2, w):
    n, v = x2.shape
    grid = n // _BR
    out = pl.pallas_call(
        _tc_body,
        grid=(grid,),
        in_specs=[
            pl.BlockSpec((_BR, v), lambda i: (i, 0)),
            pl.BlockSpec((_BR, 1), lambda i: (i, 0)),
        ],
        out_specs=pl.BlockSpec((1, 1), lambda i: (0, 0)),
        out_shape=jax.ShapeDtypeStruct((1, 1), jnp.float32),
    )(x2, w)
    return out[0, 0]


@functools.lru_cache(maxsize=None)
def _sc_gather_build(n, v):
    info = plsc.get_sparse_core_info()
    nc, ns, L = info.num_cores, info.num_subcores, info.num_lanes
    nw = nc * ns
    bpw = n // nw
    mesh = plsc.VectorSubcoreMesh(core_axis_name="c", subcore_axis_name="s")

    @functools.partial(
        pl.kernel,
        mesh=mesh,
        compiler_params=pltpu.CompilerParams(use_tc_tiling_on_sc=True),
        out_type=[
            jax.ShapeDtypeStruct((nw, L), jnp.float32),
            jax.ShapeDtypeStruct((nw, L), jnp.float32),
        ],
        scratch_types=[
            pltpu.VMEM((bpw,), jnp.int32),
            pltpu.VMEM((8, 128), jnp.float32),
            pltpu.VMEM((L,), jnp.float32),
            pltpu.VMEM((L,), jnp.float32),
        ],
    )
    def sc_gather(x_hbm, tgt_hbm, gsum_hbm, cnt_hbm,
                  tgt_v, buf_v, g_v, c_v):
        wid = lax.axis_index("s") * nc + lax.axis_index("c")
        base = wid * bpw
        pltpu.sync_copy(tgt_hbm.at[pl.ds(base, bpw)], tgt_v)
        lane = lax.iota(jnp.int32, L)
        g = jnp.zeros((L,), jnp.float32)
        c = jnp.zeros((L,), jnp.float32)
        for j in range(bpw // L):
            t16 = tgt_v[pl.ds(j * L, L)]
            c = c + jnp.where(t16 != _PAD, 1.0, 0.0)
        for i in range(bpw):
            t16 = tgt_v[pl.ds((i // L) * L, L)]
            ti = t16[i % L]
            ctile = pl.multiple_of((ti >> 7) << 7, 128)
            rtile = pl.multiple_of(base + (i // 8) * 8, 8)
            pltpu.sync_copy(
                x_hbm.at[pl.ds(rtile, 8), pl.ds(ctile, 128)], buf_v)
            cseg = ((ti >> 4) << 4) - ctile
            seg = buf_v[i % 8, pl.ds(cseg, L)]
            hit = jnp.where(lane == (ti & (L - 1)), seg, 0.0)
            validf = jnp.minimum(ti, 1).astype(jnp.float32)
            g = g + hit * validf
        g_v[...] = g
        c_v[...] = c
        pltpu.sync_copy(g_v, gsum_hbm.at[wid])
        pltpu.sync_copy(c_v, cnt_hbm.at[wid])

    return sc_gather


def kernel(x, target):
    v = x.shape[-1]
    x2 = x.reshape(-1, v)
    n = x2.shape[0]
    t = target.reshape(-1)

    eps = _SMOOTHING / (v - 2)
    c_row = (v - 2) * eps * math.log(eps) + _CONF * math.log(_CONF)

    w = (t != _PAD).astype(jnp.float32).reshape(n, 1)

    gpart, cpart = _sc_gather_build(n, v)(x2, t)
    tcsum = _tc_rowsum(x2, w)

    gsum = jnp.sum(gpart)
    nval = jnp.sum(cpart)
    return nval * c_row - eps * tcsum - (_CONF - eps) * gsum


# speedup vs baseline: 7.6351x; 3.1886x over previous
import functools
import math

import jax
import jax.numpy as jnp
from jax import lax
from jax.experimental import pallas as pl
from jax.experimental.pallas import tpu as pltpu
from jax.experimental.pallas import tpu_sc as plsc

_PAD = 0
_SMOOTHING = 0.1
_CONF = 1.0 - _SMOOTHING

_BV = 1000


def _tc_body(x_ref, sum_ref, x0_ref):
    j = pl.program_id(0)
    xb = x_ref[...]
    part = jnp.sum(xb, axis=0, keepdims=True)

    @pl.when(j == 0)
    def _init():
        sum_ref[...] = jnp.zeros_like(sum_ref)
        x0_ref[...] = xb[0:1, :]

    sum_ref[...] += part


def _tc_colsum(xt):
    v, n = xt.shape
    grid = v // _BV
    return pl.pallas_call(
        _tc_body,
        grid=(grid,),
        in_specs=[pl.BlockSpec((_BV, n), lambda j: (j, 0))],
        out_specs=[
            pl.BlockSpec((1, n), lambda j: (0, 0)),
            pl.BlockSpec((1, n), lambda j: (0, 0)),
        ],
        out_shape=[
            jax.ShapeDtypeStruct((1, n), jnp.float32),
            jax.ShapeDtypeStruct((1, n), jnp.float32),
        ],
    )(xt)


@functools.lru_cache(maxsize=None)
def _sc_gather_build(n, v):
    info = plsc.get_sparse_core_info()
    nc, ns, L = info.num_cores, info.num_subcores, info.num_lanes
    nw = nc * ns
    bpw = n // nw
    mesh = plsc.VectorSubcoreMesh(core_axis_name="c", subcore_axis_name="s")

    @functools.partial(
        pl.kernel,
        mesh=mesh,
        compiler_params=pltpu.CompilerParams(use_tc_tiling_on_sc=True),
        out_type=[
            jax.ShapeDtypeStruct((nw, L), jnp.float32),
            jax.ShapeDtypeStruct((nw, L), jnp.float32),
        ],
        scratch_types=[
            pltpu.VMEM((bpw,), jnp.int32),
            pltpu.VMEM((8, 128), jnp.float32),
            pltpu.VMEM((L,), jnp.float32),
            pltpu.VMEM((L,), jnp.float32),
        ],
    )
    def sc_gather(xt_hbm, tgt_hbm, gsum_hbm, cnt_hbm,
                  tgt_v, buf_v, g_v, c_v):
        wid = lax.axis_index("s") * nc + lax.axis_index("c")
        base = wid * bpw
        pltpu.sync_copy(tgt_hbm.at[pl.ds(base, bpw)], tgt_v)
        lane = lax.iota(jnp.int32, L)
        g = jnp.zeros((L,), jnp.float32)
        c = jnp.zeros((L,), jnp.float32)
        for j in range(bpw // L):
            t16 = tgt_v[pl.ds(j * L, L)]
            c = c + jnp.where(t16 != _PAD, 1.0, 0.0)
        for i in range(bpw):
            t16 = tgt_v[pl.ds((i // L) * L, L)]
            ti = t16[i % L]
            nabs = base + i
            rtile = pl.multiple_of((ti >> 3) << 3, 8)
            ctile = pl.multiple_of((nabs >> 7) << 7, 128)
            pltpu.sync_copy(
                xt_hbm.at[pl.ds(rtile, 8), pl.ds(ctile, 128)], buf_v)
            nloc = nabs - ctile
            cseg = (nloc >> 4) << 4
            rr = ti & 7
            hit = jnp.zeros((L,), jnp.float32)
            for r in range(8):
                seg = buf_v[r, pl.ds(cseg, L)]
                coef = (1 - jnp.minimum(jnp.abs(rr - r), 1)).astype(
                    jnp.float32)
                hit = hit + seg * coef
            sel = jnp.where(lane == (nloc & (L - 1)), hit, 0.0)
            validf = jnp.minimum(ti, 1).astype(jnp.float32)
            g = g + sel * validf
        g_v[...] = g
        c_v[...] = c
        pltpu.sync_copy(g_v, gsum_hbm.at[wid])
        pltpu.sync_copy(c_v, cnt_hbm.at[wid])

    return sc_gather


def kernel(x, target):
    v = x.shape[-1]
    x2 = x.reshape(-1, v)
    n = x2.shape[0]
    t = target.reshape(-1)
    xt = jnp.swapaxes(x2, 0, 1)

    eps = _SMOOTHING / (v - 2)
    c_row = (v - 2) * eps * math.log(eps) + _CONF * math.log(_CONF)

    gpart, cpart = _sc_gather_build(n, v)(xt, t)
    sums, x0row = _tc_colsum(xt)

    wf = (t != _PAD).astype(jnp.float32)
    tcsum = jnp.dot(sums[0] - x0row[0], wf)
    gsum = jnp.sum(gpart)
    nval = jnp.sum(cpart)
    return nval * c_row - eps * tcsum - (_CONF - eps) * gsum


# speedup vs baseline: 7.6365x; 1.0002x over previous
import functools
import math

import jax
import jax.numpy as jnp
from jax import lax
from jax.experimental import pallas as pl
from jax.experimental.pallas import tpu as pltpu
from jax.experimental.pallas import tpu_sc as plsc

_PAD = 0
_SMOOTHING = 0.1
_CONF = 1.0 - _SMOOTHING

_BV = 2000


def _tc_body(x_ref, sum_ref, x0_ref):
    j = pl.program_id(0)
    xb = x_ref[...]
    part = jnp.sum(xb, axis=0, keepdims=True)

    @pl.when(j == 0)
    def _init():
        sum_ref[...] = jnp.zeros_like(sum_ref)
        x0_ref[...] = xb[0:1, :]

    sum_ref[...] += part


def _tc_colsum(xt):
    v, n = xt.shape
    grid = v // _BV
    return pl.pallas_call(
        _tc_body,
        grid=(grid,),
        in_specs=[pl.BlockSpec((_BV, n), lambda j: (j, 0))],
        out_specs=[
            pl.BlockSpec((1, n), lambda j: (0, 0)),
            pl.BlockSpec((1, n), lambda j: (0, 0)),
        ],
        out_shape=[
            jax.ShapeDtypeStruct((1, n), jnp.float32),
            jax.ShapeDtypeStruct((1, n), jnp.float32),
        ],
    )(xt)


@functools.lru_cache(maxsize=None)
def _sc_gather_build(n, v):
    info = plsc.get_sparse_core_info()
    nc, ns, L = info.num_cores, info.num_subcores, info.num_lanes
    nw = nc * ns
    bpw = n // nw
    mesh = plsc.VectorSubcoreMesh(core_axis_name="c", subcore_axis_name="s")

    @functools.partial(
        pl.kernel,
        mesh=mesh,
        compiler_params=pltpu.CompilerParams(use_tc_tiling_on_sc=True),
        out_type=[
            jax.ShapeDtypeStruct((nw, L), jnp.float32),
            jax.ShapeDtypeStruct((nw, L), jnp.float32),
        ],
        scratch_types=[
            pltpu.VMEM((bpw,), jnp.int32),
            pltpu.VMEM((8, 128), jnp.float32),
            pltpu.VMEM((L,), jnp.float32),
            pltpu.VMEM((L,), jnp.float32),
        ],
    )
    def sc_gather(xt_hbm, tgt_hbm, gsum_hbm, cnt_hbm,
                  tgt_v, buf_v, g_v, c_v):
        wid = lax.axis_index("s") * nc + lax.axis_index("c")
        base = wid * bpw
        pltpu.sync_copy(tgt_hbm.at[pl.ds(base, bpw)], tgt_v)
        lane = lax.iota(jnp.int32, L)
        g = jnp.zeros((L,), jnp.float32)
        c = jnp.zeros((L,), jnp.float32)
        for j in range(bpw // L):
            t16 = tgt_v[pl.ds(j * L, L)]
            c = c + jnp.where(t16 != _PAD, 1.0, 0.0)
        for i in range(bpw):
            t16 = tgt_v[pl.ds((i // L) * L, L)]
            ti = t16[i % L]
            nabs = base + i
            rtile = pl.multiple_of((ti >> 3) << 3, 8)
            ctile = pl.multiple_of((nabs >> 7) << 7, 128)
            pltpu.sync_copy(
                xt_hbm.at[pl.ds(rtile, 8), pl.ds(ctile, 128)], buf_v)
            nloc = nabs - ctile
            cseg = (nloc >> 4) << 4
            rr = ti & 7
            hit = jnp.zeros((L,), jnp.float32)
            for r in range(8):
                seg = buf_v[r, pl.ds(cseg, L)]
                coef = (1 - jnp.minimum(jnp.abs(rr - r), 1)).astype(
                    jnp.float32)
                hit = hit + seg * coef
            sel = jnp.where(lane == (nloc & (L - 1)), hit, 0.0)
            validf = jnp.minimum(ti, 1).astype(jnp.float32)
            g = g + sel * validf
        g_v[...] = g
        c_v[...] = c
        pltpu.sync_copy(g_v, gsum_hbm.at[wid])
        pltpu.sync_copy(c_v, cnt_hbm.at[wid])

    return sc_gather


def kernel(x, target):
    v = x.shape[-1]
    x2 = x.reshape(-1, v)
    n = x2.shape[0]
    t = target.reshape(-1)
    xt = jnp.swapaxes(x2, 0, 1)

    eps = _SMOOTHING / (v - 2)
    c_row = (v - 2) * eps * math.log(eps) + _CONF * math.log(_CONF)

    gpart, cpart = _sc_gather_build(n, v)(xt, t)
    sums, x0row = _tc_colsum(xt)

    wf = (t != _PAD).astype(jnp.float32)
    tcsum = jnp.dot(sums[0] - x0row[0], wf)
    gsum = jnp.sum(gpart)
    nval = jnp.sum(cpart)
    return nval * c_row - eps * tcsum - (_CONF - eps) * gsum
